# baseline (speedup 1.0000x reference)
import functools

import jax
import jax.numpy as jnp
from jax import lax
from jax.experimental import pallas as pl
from jax.experimental.pallas import tpu as pltpu
from jax.experimental.pallas import tpu_sc as plsc

NC, NS = 2, 16
NW = NC * NS
B = 16384
D = 64
L = 16
BPW = B // NW
NCHUNK = 4
CH = BPW // NCHUNK
IDX_ROWS = B // 128 // NW


def _sc_dot_body(u_hbm, i_hbm, ut_hbm, it_hbm, out_hbm,
                 uidx, iidx, urows, irows, wbuf, outv, sem):
    wid = lax.axis_index("s") * NC + lax.axis_index("c")
    row0 = wid * IDX_ROWS

    pltpu.sync_copy(u_hbm.at[pl.ds(row0, IDX_ROWS)], uidx)
    pltpu.sync_copy(i_hbm.at[pl.ds(row0, IDX_ROWS)], iidx)

    copies = []
    for q in range(NCHUNK):
        copies.append(pltpu.async_copy(
            ut_hbm.at[uidx.at[q]], urows.at[pl.ds(q * CH, CH)], sem))
        copies.append(pltpu.async_copy(
            it_hbm.at[iidx.at[q]], irows.at[pl.ds(q * CH, CH)], sem))
    for c in copies:
        c.wait()

    def row_body(r, carry):
        w = urows[r, pl.ds(0, L)] * irows[r, pl.ds(0, L)]
        w += urows[r, pl.ds(L, L)] * irows[r, pl.ds(L, L)]
        w += urows[r, pl.ds(2 * L, L)] * irows[r, pl.ds(2 * L, L)]
        w += urows[r, pl.ds(3 * L, L)] * irows[r, pl.ds(3 * L, L)]
        wbuf[pl.ds(r * L, L)] = w
        return carry
    lax.fori_loop(0, BPW, row_body, 0)

    def grp_body(g, carry):
        j0 = g * L
        base_ids = (j0 + lax.iota(jnp.int32, L)) * L
        acc = plsc.load_gather(wbuf, [base_ids])
        for l in range(1, L):
            acc += plsc.load_gather(wbuf, [base_ids + l])
        outv[pl.ds(j0, L)] = acc
        return carry
    lax.fori_loop(0, BPW // L, grp_body, 0)

    pltpu.sync_copy(outv, out_hbm.at[pl.ds(wid * BPW, BPW)])


def kernel(user, item, user_table, item_table):
    user2d = user.reshape(128, 128)
    item2d = item.reshape(128, 128)
    mesh = plsc.VectorSubcoreMesh(core_axis_name="c", subcore_axis_name="s")
    out = pl.kernel(
        _sc_dot_body,
        out_type=jax.ShapeDtypeStruct((B,), jnp.float32),
        mesh=mesh,
        compiler_params=pltpu.CompilerParams(
            needs_layout_passes=False, use_tc_tiling_on_sc=False),
        scratch_types=[
            pltpu.VMEM((NCHUNK, CH), jnp.int32),
            pltpu.VMEM((NCHUNK, CH), jnp.int32),
            pltpu.VMEM((BPW, D), jnp.float32),
            pltpu.VMEM((BPW, D), jnp.float32),
            pltpu.VMEM((BPW * L,), jnp.float32),
            pltpu.VMEM((BPW,), jnp.float32),
            pltpu.SemaphoreType.DMA,
        ],
    )(user2d, item2d, user_table, item_table)
    return out


# speedup vs baseline: 1.5322x; 1.5322x over previous
import functools

import jax
import jax.numpy as jnp
from jax import lax
from jax.experimental import pallas as pl
from jax.experimental.pallas import tpu as pltpu
from jax.experimental.pallas import tpu_sc as plsc

NC, NS = 2, 16
NW = NC * NS
B = 16384
D = 64
L = 16
BPW = B // NW
IDX_ROWS = B // 128 // NW
NCHUNK = BPW // L


def _sc_dot_body(u_hbm, i_hbm, ut_hbm, it_hbm, out_hbm,
                 uidx, iidx, ubuf, ibuf, wbuf, outv, sem):
    wid = lax.axis_index("s") * NC + lax.axis_index("c")
    row0 = wid * IDX_ROWS

    pltpu.sync_copy(u_hbm.at[pl.ds(row0, IDX_ROWS)], uidx)
    pltpu.sync_copy(i_hbm.at[pl.ds(row0, IDX_ROWS)], iidx)

    def chunk_body(t, carry):
        uv = uidx[t // 8, pl.ds((t % 8) * L, L)]
        iv = iidx[t // 8, pl.ds((t % 8) * L, L)]
        cps = []
        for j in range(L):
            cps.append(pltpu.async_copy(ut_hbm.at[uv[j]], ubuf.at[j], sem))
            cps.append(pltpu.async_copy(it_hbm.at[iv[j]], ibuf.at[j], sem))
        for c in cps:
            c.wait()
        for j in range(L):
            w = ubuf[j, pl.ds(0, L)] * ibuf[j, pl.ds(0, L)]
            w += ubuf[j, pl.ds(L, L)] * ibuf[j, pl.ds(L, L)]
            w += ubuf[j, pl.ds(2 * L, L)] * ibuf[j, pl.ds(2 * L, L)]
            w += ubuf[j, pl.ds(3 * L, L)] * ibuf[j, pl.ds(3 * L, L)]
            wbuf[pl.ds((t * L + j) * L, L)] = w
        return carry

    lax.fori_loop(0, NCHUNK, chunk_body, 0)

    def grp_body(g, carry):
        j0 = g * L
        base_ids = (j0 + lax.iota(jnp.int32, L)) * L
        acc = plsc.load_gather(wbuf, [base_ids])
        for l in range(1, L):
            acc += plsc.load_gather(wbuf, [base_ids + l])
        outv[pl.ds(j0, L)] = acc
        return carry
    lax.fori_loop(0, BPW // L, grp_body, 0)

    pltpu.sync_copy(outv, out_hbm.at[pl.ds(wid * BPW, BPW)])


def kernel(user, item, user_table, item_table):
    user2d = user.reshape(128, 128)
    item2d = item.reshape(128, 128)
    mesh = plsc.VectorSubcoreMesh(core_axis_name="c", subcore_axis_name="s")
    out = pl.kernel(
        _sc_dot_body,
        out_type=jax.ShapeDtypeStruct((B,), jnp.float32),
        mesh=mesh,
        compiler_params=pltpu.CompilerParams(needs_layout_passes=False),
        scratch_types=[
            pltpu.VMEM((IDX_ROWS, 128), jnp.int32),
            pltpu.VMEM((IDX_ROWS, 128), jnp.int32),
            pltpu.VMEM((L, D), jnp.float32),
            pltpu.VMEM((L, D), jnp.float32),
            pltpu.VMEM((BPW * L,), jnp.float32),
            pltpu.VMEM((BPW,), jnp.float32),
            pltpu.SemaphoreType.DMA,
        ],
    )(user2d, item2d, user_table, item_table)
    return out
